# Initial kernel scaffold; baseline (speedup 1.0000x reference)
import jax
import jax.numpy as jnp
from jax.experimental import pallas as pl


def kernel(users, edge_index, edge_weight, user_table, item_table, aspect_emb, W_mlp, b_mlp, user_aspect_idx, item_aspect_idx):
    raise NotImplementedError("write your pallas kernel here")


# speedup vs baseline: 2.9872x; 2.9872x over previous
import functools

import jax
import jax.numpy as jnp
from jax import lax
from jax.experimental import pallas as pl
from jax.experimental.pallas import tpu as pltpu
import jax.experimental.pallas.tpu_sc as plsc

N_USERS = 25000
N_ITEMS = 25000
NN = N_USERS + N_ITEMS
D = 64
E = 800000
NQ = 512
B_USERS = 1024

NC = 2
NS = 16
NW = NC * NS
L = 16

CH = 128
NCHUNK = E // CH
EITERS = -(-NCHUNK // NS)

HALF = NN // NC
ACC_ROWS = 25088
RPT = ACC_ROWS // NS
LAST_VALID = HALF - (NS - 1) * RPT
ZR = 98

@functools.lru_cache(maxsize=None)
def _mesh():
    return plsc.VectorSubcoreMesh(
        core_axis_name="c", subcore_axis_name="s",
        num_cores=NC, num_subcores=NS)


def _spmm_body(emb, src, dst, w, out, srcv, dstv, wv, rows, zbuf, acc, sem):
    c = lax.axis_index("c")
    s = lax.axis_index("s")
    lo = c * HALF

    zro = jnp.zeros((L,), jnp.float32)

    def _zb(i, carry):
        for q in range(D // L):
            zbuf[i, pl.ds(q * L, L)] = zro
        return carry

    lax.fori_loop(0, ZR, _zb, 0)
    for t in range(RPT // ZR):
        pltpu.sync_copy(zbuf, acc.at[pl.ds(s * RPT + t * ZR, ZR)])
    plsc.subcore_barrier()

    def _step(k, carry):
        base = (k * NS + s) * CH

        @pl.when(base < E)
        def _():
            pltpu.sync_copy(src.at[pl.ds(base, CH)], srcv)
            pltpu.sync_copy(dst.at[pl.ds(base, CH)], dstv)
            pltpu.sync_copy(w.at[pl.ds(base, CH)], wv)
            pltpu.async_copy(emb.at[srcv], rows, sem).wait()
            for j in range(CH // L):
                d16 = dstv[pl.ds(j * L, L)]
                inb = (d16 >= lo) & (d16 < lo + HALF)
                dstv[pl.ds(j * L, L)] = jnp.where(inb, d16 - lo, HALF)
                w16 = wv[pl.ds(j * L, L)]
                for e in range(L):
                    r = j * L + e
                    wb = w16.at[jnp.full((L,), e, jnp.int32)].get(
                        mode="promise_in_bounds")
                    for q in range(D // L):
                        rows[r, pl.ds(q * L, L)] = rows[r, pl.ds(q * L, L)] * wb
            pltpu.sync_copy(rows, acc.at[dstv], add=True)

        return carry

    lax.fori_loop(0, EITERS, _step, 0)
    plsc.subcore_barrier()

    @pl.when(s < NS - 1)
    def _():
        pltpu.sync_copy(acc.at[pl.ds(s * RPT, RPT)],
                        out.at[pl.ds(lo + s * RPT, RPT)])

    @pl.when(s == NS - 1)
    def _():
        pltpu.sync_copy(acc.at[pl.ds((NS - 1) * RPT, LAST_VALID)],
                        out.at[pl.ds(lo + (NS - 1) * RPT, LAST_VALID)])


@functools.lru_cache(maxsize=None)
def _spmm_kernel():
    return pl.kernel(
        _spmm_body,
        out_type=jax.ShapeDtypeStruct((NN, D), jnp.float32),
        mesh=_mesh(),
        scratch_types=[
            pltpu.VMEM((CH,), jnp.int32),
            pltpu.VMEM((CH,), jnp.int32),
            pltpu.VMEM((CH,), jnp.float32),
            pltpu.VMEM((CH, D), jnp.float32),
            pltpu.VMEM((ZR, D), jnp.float32),
            pltpu.VMEM_SHARED((ACC_ROWS, D), jnp.float32),
            pltpu.SemaphoreType.DMA,
        ],
        compiler_params=pltpu.CompilerParams(use_tc_tiling_on_sc=False),
    )


UPT = B_USERS // NW
CI = 40
NIC = N_ITEMS // CI
IITERS = -(-NIC // NW)


def _build_body(light, asp, users, uidx, iidx, uout, iout,
                uv, uix, uacc, iix, iacc, sem):
    c = lax.axis_index("c")
    s = lax.axis_index("s")
    wid = s * NC + c

    ub = wid * UPT
    pltpu.sync_copy(users.at[pl.ds(ub, UPT)], uv)
    pltpu.sync_copy(uidx.at[pl.ds(ub, UPT)], uix)
    pltpu.async_copy(light.at[uv], uacc, sem).wait()
    pltpu.async_copy(asp.at[uix], uacc, sem, add=True).wait()
    pltpu.sync_copy(uacc, uout.at[pl.ds(ub, UPT)])

    def _step(k, carry):
        g = k * NW + wid

        @pl.when(g < NIC)
        def _():
            b = g * CI
            pltpu.sync_copy(light.at[pl.ds(N_USERS + b, CI)], iacc)
            pltpu.sync_copy(iidx.at[pl.ds(b, CI)], iix)
            pltpu.async_copy(asp.at[iix], iacc, sem, add=True).wait()
            pltpu.sync_copy(iacc, iout.at[pl.ds(b, CI)])

        return carry

    lax.fori_loop(0, IITERS, _step, 0)


@functools.lru_cache(maxsize=None)
def _build_kernel():
    return pl.kernel(
        _build_body,
        out_type=(jax.ShapeDtypeStruct((B_USERS, D), jnp.float32),
                  jax.ShapeDtypeStruct((N_ITEMS, D), jnp.float32)),
        mesh=_mesh(),
        scratch_types=[
            pltpu.VMEM((UPT,), jnp.int32),
            pltpu.VMEM((UPT,), jnp.int32),
            pltpu.VMEM((UPT, D), jnp.float32),
            pltpu.VMEM((CI,), jnp.int32),
            pltpu.VMEM((CI, D), jnp.float32),
            pltpu.SemaphoreType.DMA,
        ],
        compiler_params=pltpu.CompilerParams(use_tc_tiling_on_sc=False),
    )


MROWS = NN * D // 128
MB = 1000


def _mean_mlp_body(e0, e1, e2, e3, aspw, wm, bm, light, asp64):
    light[...] = (e0[...] + e1[...] + e2[...] + e3[...]) * 0.25

    @pl.when(pl.program_id(0) == 0)
    def _():
        asp64[...] = jnp.dot(aspw[...], wm[...],
                             preferred_element_type=jnp.float32) + bm[...]


def _mean_mlp(e0, e1, e2, e3, aspw, wm, bm):
    v = lambda x: x.reshape(MROWS, 128)
    light, asp64 = pl.pallas_call(
        _mean_mlp_body,
        grid=(MROWS // MB,),
        in_specs=[
            pl.BlockSpec((MB, 128), lambda i: (i, 0)),
            pl.BlockSpec((MB, 128), lambda i: (i, 0)),
            pl.BlockSpec((MB, 128), lambda i: (i, 0)),
            pl.BlockSpec((MB, 128), lambda i: (i, 0)),
            pl.BlockSpec((NQ, 384), lambda i: (0, 0)),
            pl.BlockSpec((384, D), lambda i: (0, 0)),
            pl.BlockSpec((1, D), lambda i: (0, 0)),
        ],
        out_specs=[
            pl.BlockSpec((MB, 128), lambda i: (i, 0)),
            pl.BlockSpec((NQ, D), lambda i: (0, 0)),
        ],
        out_shape=[
            jax.ShapeDtypeStruct((MROWS, 128), jnp.float32),
            jax.ShapeDtypeStruct((NQ, D), jnp.float32),
        ],
    )(v(e0), v(e1), v(e2), v(e3), aspw, wm, bm.reshape(1, D))
    return light.reshape(NN, D), asp64


IB = 512


def _rating_body(u, it, out):
    acc = lax.dot_general(u[...], it[...], (((1,), (1,)), ((), ())),
                          preferred_element_type=jnp.float32)
    out[...] = jax.nn.sigmoid(acc)


def _rating(users_emb, items_emb):
    return pl.pallas_call(
        _rating_body,
        grid=(-(-N_ITEMS // IB),),
        in_specs=[
            pl.BlockSpec((B_USERS, D), lambda i: (0, 0)),
            pl.BlockSpec((IB, D), lambda i: (i, 0)),
        ],
        out_specs=pl.BlockSpec((B_USERS, IB), lambda i: (0, i)),
        out_shape=jax.ShapeDtypeStruct((B_USERS, N_ITEMS), jnp.float32),
    )(users_emb, items_emb)


def kernel(users, edge_index, edge_weight, user_table, item_table,
           aspect_emb, W_mlp, b_mlp, user_aspect_idx, item_aspect_idx):
    users = users.astype(jnp.int32)
    src = edge_index[0]
    dst = edge_index[1]
    e0 = jnp.concatenate([user_table, item_table], axis=0)
    spmm = _spmm_kernel()
    e1 = spmm(e0, src, dst, edge_weight)
    e2 = spmm(e1, src, dst, edge_weight)
    e3 = spmm(e2, src, dst, edge_weight)
    light, asp64 = _mean_mlp(e0, e1, e2, e3, aspect_emb, W_mlp, b_mlp)
    uidx = jnp.take(user_aspect_idx, users)
    users_emb, items_emb = _build_kernel()(
        light, asp64, users, uidx, item_aspect_idx)
    return _rating(users_emb, items_emb)


# speedup vs baseline: 5.4082x; 1.8105x over previous
import functools

import jax
import jax.numpy as jnp
from jax import lax
from jax.experimental import pallas as pl
from jax.experimental.pallas import tpu as pltpu
import jax.experimental.pallas.tpu_sc as plsc

N_USERS = 25000
N_ITEMS = 25000
NN = N_USERS + N_ITEMS
D = 64
E = 800000
NQ = 512
B_USERS = 1024

NC = 2
NS = 16
NW = NC * NS
L = 16

CH = 80
EPT = E // NS
NCPG = 5
GEDGES = NCPG * CH
NG = EPT // GEDGES

HALF = NN // NC
ACC_ROWS = 25088
RPT = ACC_ROWS // NS
LAST_VALID = HALF - (NS - 1) * RPT
ZR = 28

@functools.lru_cache(maxsize=None)
def _mesh():
    return plsc.VectorSubcoreMesh(
        core_axis_name="c", subcore_axis_name="s",
        num_cores=NC, num_subcores=NS)


def _spmm_body(emb, src, dst, w, out, bsrc, bdst, bw, dstl, rows, zbuf, acc,
               esem, gsem, ssem, zsem):
    c = lax.axis_index("c")
    s = lax.axis_index("s")
    lo = c * HALF

    zro = jnp.zeros((L,), jnp.float32)

    def _zb(i, carry):
        for q in range(D // L):
            zbuf[i, pl.ds(q * L, L)] = zro
        return carry

    lax.fori_loop(0, ZR, _zb, 0)

    def _zfire(t, carry):
        pltpu.async_copy(zbuf, acc.at[pl.ds(s * RPT + t * ZR, ZR)], zsem)
        return carry

    def _zdrain(t, carry):
        pltpu.make_async_copy(
            zbuf, acc.at[pl.ds(s * RPT + t * ZR, ZR)], zsem).wait()
        return carry

    lax.fori_loop(0, RPT // ZR, _zfire, 0)
    lax.fori_loop(0, RPT // ZR, _zdrain, 0)
    plsc.subcore_barrier()

    def _eload(g, slot, fire):
        hb = s * EPT + g * GEDGES
        for buf, tab in ((bsrc, src), (bdst, dst), (bw, w)):
            if fire:
                pltpu.async_copy(tab.at[pl.ds(hb, GEDGES)], buf.at[slot],
                                 esem)
            else:
                pltpu.make_async_copy(tab.at[pl.ds(hb, GEDGES)], buf.at[slot],
                                      esem).wait()

    _eload(0, 0, True)

    def _group(g, carry):
        slot = lax.rem(g, 2)
        _eload(g, slot, False)

        @pl.when(g + 1 < NG)
        def _():
            _eload(g + 1, 1 - slot, True)

        @pl.when(g > 0)
        def _():
            for i in range(NCPG):
                pltpu.make_async_copy(
                    rows.at[i], acc.at[dstl.at[i]], ssem).wait()

        descs = [
            pltpu.async_copy(emb.at[bsrc.at[slot, pl.ds(i * CH, CH)]],
                             rows.at[i], gsem)
            for i in range(NCPG)
        ]
        for i in range(NCPG):
            descs[i].wait()
            for j in range(CH // L):
                o = i * CH + j * L
                d16 = bdst[slot, pl.ds(o, L)]
                inb = (d16 >= lo) & (d16 < lo + HALF)
                dstl[i, pl.ds(j * L, L)] = jnp.where(inb, d16 - lo, HALF)
                w16 = bw[slot, pl.ds(o, L)]
                for e in range(L):
                    r = j * L + e
                    wb = w16.at[jnp.full((L,), e, jnp.int32)].get(
                        mode="promise_in_bounds")
                    for q in range(D // L):
                        rows[i, r, pl.ds(q * L, L)] = (
                            rows[i, r, pl.ds(q * L, L)] * wb)
            pltpu.async_copy(rows.at[i], acc.at[dstl.at[i]], ssem, add=True)
        return carry

    lax.fori_loop(0, NG, _group, 0)
    for i in range(NCPG):
        pltpu.make_async_copy(rows.at[i], acc.at[dstl.at[i]], ssem).wait()
    plsc.subcore_barrier()

    @pl.when(s < NS - 1)
    def _():
        pltpu.sync_copy(acc.at[pl.ds(s * RPT, RPT)],
                        out.at[pl.ds(lo + s * RPT, RPT)])

    @pl.when(s == NS - 1)
    def _():
        pltpu.sync_copy(acc.at[pl.ds((NS - 1) * RPT, LAST_VALID)],
                        out.at[pl.ds(lo + (NS - 1) * RPT, LAST_VALID)])


@functools.lru_cache(maxsize=None)
def _spmm_kernel():
    return pl.kernel(
        _spmm_body,
        out_type=jax.ShapeDtypeStruct((NN, D), jnp.float32),
        mesh=_mesh(),
        scratch_types=[
            pltpu.VMEM((2, GEDGES), jnp.int32),
            pltpu.VMEM((2, GEDGES), jnp.int32),
            pltpu.VMEM((2, GEDGES), jnp.float32),
            pltpu.VMEM((NCPG, CH), jnp.int32),
            pltpu.VMEM((NCPG, CH, D), jnp.float32),
            pltpu.VMEM((ZR, D), jnp.float32),
            pltpu.VMEM_SHARED((ACC_ROWS, D), jnp.float32),
            pltpu.SemaphoreType.DMA,
            pltpu.SemaphoreType.DMA,
            pltpu.SemaphoreType.DMA,
            pltpu.SemaphoreType.DMA,
        ],
        compiler_params=pltpu.CompilerParams(use_tc_tiling_on_sc=False),
    )


UPT = B_USERS // NW
CI = 40
NIC = N_ITEMS // CI
IITERS = -(-NIC // NW)


def _build_body(light, asp, users, uidx, iidx, uout, iout,
                uv, uix, uacc, iix, iacc, sem):
    c = lax.axis_index("c")
    s = lax.axis_index("s")
    wid = s * NC + c

    ub = wid * UPT
    pltpu.sync_copy(users.at[pl.ds(ub, UPT)], uv)
    pltpu.sync_copy(uidx.at[pl.ds(ub, UPT)], uix)
    pltpu.async_copy(light.at[uv], uacc, sem).wait()
    pltpu.async_copy(asp.at[uix], uacc, sem, add=True).wait()
    pltpu.sync_copy(uacc, uout.at[pl.ds(ub, UPT)])

    def _step(k, carry):
        g = k * NW + wid

        @pl.when(g < NIC)
        def _():
            b = g * CI
            pltpu.sync_copy(light.at[pl.ds(N_USERS + b, CI)], iacc)
            pltpu.sync_copy(iidx.at[pl.ds(b, CI)], iix)
            pltpu.async_copy(asp.at[iix], iacc, sem, add=True).wait()
            pltpu.sync_copy(iacc, iout.at[pl.ds(b, CI)])

        return carry

    lax.fori_loop(0, IITERS, _step, 0)


@functools.lru_cache(maxsize=None)
def _build_kernel():
    return pl.kernel(
        _build_body,
        out_type=(jax.ShapeDtypeStruct((B_USERS, D), jnp.float32),
                  jax.ShapeDtypeStruct((N_ITEMS, D), jnp.float32)),
        mesh=_mesh(),
        scratch_types=[
            pltpu.VMEM((UPT,), jnp.int32),
            pltpu.VMEM((UPT,), jnp.int32),
            pltpu.VMEM((UPT, D), jnp.float32),
            pltpu.VMEM((CI,), jnp.int32),
            pltpu.VMEM((CI, D), jnp.float32),
            pltpu.SemaphoreType.DMA,
        ],
        compiler_params=pltpu.CompilerParams(use_tc_tiling_on_sc=False),
    )


MROWS = NN * D // 128
MB = 1000


def _mean_mlp_body(e0, e1, e2, e3, aspw, wm, bm, light, asp64):
    light[...] = (e0[...] + e1[...] + e2[...] + e3[...]) * 0.25

    @pl.when(pl.program_id(0) == 0)
    def _():
        asp64[...] = jnp.dot(aspw[...], wm[...],
                             preferred_element_type=jnp.float32) + bm[...]


def _mean_mlp(e0, e1, e2, e3, aspw, wm, bm):
    v = lambda x: x.reshape(MROWS, 128)
    light, asp64 = pl.pallas_call(
        _mean_mlp_body,
        grid=(MROWS // MB,),
        in_specs=[
            pl.BlockSpec((MB, 128), lambda i: (i, 0)),
            pl.BlockSpec((MB, 128), lambda i: (i, 0)),
            pl.BlockSpec((MB, 128), lambda i: (i, 0)),
            pl.BlockSpec((MB, 128), lambda i: (i, 0)),
            pl.BlockSpec((NQ, 384), lambda i: (0, 0)),
            pl.BlockSpec((384, D), lambda i: (0, 0)),
            pl.BlockSpec((1, D), lambda i: (0, 0)),
        ],
        out_specs=[
            pl.BlockSpec((MB, 128), lambda i: (i, 0)),
            pl.BlockSpec((NQ, D), lambda i: (0, 0)),
        ],
        out_shape=[
            jax.ShapeDtypeStruct((MROWS, 128), jnp.float32),
            jax.ShapeDtypeStruct((NQ, D), jnp.float32),
        ],
    )(v(e0), v(e1), v(e2), v(e3), aspw, wm, bm.reshape(1, D))
    return light.reshape(NN, D), asp64


IB = 512


def _rating_body(u, it, out):
    acc = lax.dot_general(u[...], it[...], (((1,), (1,)), ((), ())),
                          preferred_element_type=jnp.float32)
    out[...] = jax.nn.sigmoid(acc)


def _rating(users_emb, items_emb):
    return pl.pallas_call(
        _rating_body,
        grid=(-(-N_ITEMS // IB),),
        in_specs=[
            pl.BlockSpec((B_USERS, D), lambda i: (0, 0)),
            pl.BlockSpec((IB, D), lambda i: (i, 0)),
        ],
        out_specs=pl.BlockSpec((B_USERS, IB), lambda i: (0, i)),
        out_shape=jax.ShapeDtypeStruct((B_USERS, N_ITEMS), jnp.float32),
    )(users_emb, items_emb)


def kernel(users, edge_index, edge_weight, user_table, item_table,
           aspect_emb, W_mlp, b_mlp, user_aspect_idx, item_aspect_idx):
    users = users.astype(jnp.int32)
    src = edge_index[0]
    dst = edge_index[1]
    e0 = jnp.concatenate([user_table, item_table], axis=0)
    spmm = _spmm_kernel()
    e1 = spmm(e0, src, dst, edge_weight)
    e2 = spmm(e1, src, dst, edge_weight)
    e3 = spmm(e2, src, dst, edge_weight)
    light, asp64 = _mean_mlp(e0, e1, e2, e3, aspect_emb, W_mlp, b_mlp)
    uidx = jnp.take(user_aspect_idx, users)
    users_emb, items_emb = _build_kernel()(
        light, asp64, users, uidx, item_aspect_idx)
    return _rating(users_emb, items_emb)


# speedup vs baseline: 5.6550x; 1.0456x over previous
import functools

import jax
import jax.numpy as jnp
from jax import lax
from jax.experimental import pallas as pl
from jax.experimental.pallas import tpu as pltpu
import jax.experimental.pallas.tpu_sc as plsc

N_USERS = 25000
N_ITEMS = 25000
NN = N_USERS + N_ITEMS
D = 64
E = 800000
NQ = 512
B_USERS = 1024

NC = 2
NS = 16
NW = NC * NS
L = 16

CH = 80
EPT = E // NS
NCPG = 5
GEDGES = NCPG * CH
NG = EPT // GEDGES

HALF = NN // NC
ACC_ROWS = 25088
RPT = ACC_ROWS // NS
LAST_VALID = HALF - (NS - 1) * RPT
ZR = 28

@functools.lru_cache(maxsize=None)
def _mesh():
    return plsc.VectorSubcoreMesh(
        core_axis_name="c", subcore_axis_name="s",
        num_cores=NC, num_subcores=NS)


def _spmm_body(emb, src, dst, w, out, bsrc, bdst, bw, dstl, rows, zbuf, acc,
               esem, gsem, ssem, zsem):
    c = lax.axis_index("c")
    s = lax.axis_index("s")
    lo = c * HALF

    zro = jnp.zeros((L,), jnp.float32)

    def _zb(i, carry):
        for q in range(D // L):
            zbuf[i, pl.ds(q * L, L)] = zro
        return carry

    lax.fori_loop(0, ZR, _zb, 0)

    def _zfire(t, carry):
        pltpu.async_copy(zbuf, acc.at[pl.ds(s * RPT + t * ZR, ZR)], zsem)
        return carry

    def _zdrain(t, carry):
        pltpu.make_async_copy(
            zbuf, acc.at[pl.ds(s * RPT + t * ZR, ZR)], zsem).wait()
        return carry

    lax.fori_loop(0, RPT // ZR, _zfire, 0)
    lax.fori_loop(0, RPT // ZR, _zdrain, 0)
    plsc.subcore_barrier()

    def _eload(g, slot, fire):
        hb = s * EPT + g * GEDGES
        for buf, tab in ((bsrc, src), (bdst, dst), (bw, w)):
            if fire:
                pltpu.async_copy(tab.at[pl.ds(hb, GEDGES)], buf.at[slot],
                                 esem)
            else:
                pltpu.make_async_copy(tab.at[pl.ds(hb, GEDGES)], buf.at[slot],
                                      esem).wait()

    _eload(0, 0, True)

    def _group(g, carry):
        slot = lax.rem(g, 2)
        _eload(g, slot, False)

        @pl.when(g + 1 < NG)
        def _():
            _eload(g + 1, 1 - slot, True)

        descs = []
        for i in range(NCPG):
            @pl.when(g > 0)
            def _(i=i):
                pltpu.make_async_copy(
                    rows.at[i], acc.at[dstl.at[i]], ssem).wait()

            descs.append(
                pltpu.async_copy(emb.at[bsrc.at[slot, pl.ds(i * CH, CH)]],
                                 rows.at[i], gsem))
        for i in range(NCPG):
            descs[i].wait()
            for j in range(CH // L):
                o = i * CH + j * L
                d16 = bdst[slot, pl.ds(o, L)]
                inb = (d16 >= lo) & (d16 < lo + HALF)
                dstl[i, pl.ds(j * L, L)] = jnp.where(inb, d16 - lo, HALF)
                w16 = bw[slot, pl.ds(o, L)]
                for e in range(L):
                    r = j * L + e
                    wb = w16.at[jnp.full((L,), e, jnp.int32)].get(
                        mode="promise_in_bounds")
                    for q in range(D // L):
                        rows[i, r, pl.ds(q * L, L)] = (
                            rows[i, r, pl.ds(q * L, L)] * wb)
            pltpu.async_copy(rows.at[i], acc.at[dstl.at[i]], ssem, add=True)
        return carry

    lax.fori_loop(0, NG, _group, 0)
    for i in range(NCPG):
        pltpu.make_async_copy(rows.at[i], acc.at[dstl.at[i]], ssem).wait()
    plsc.subcore_barrier()

    @pl.when(s < NS - 1)
    def _():
        pltpu.sync_copy(acc.at[pl.ds(s * RPT, RPT)],
                        out.at[pl.ds(lo + s * RPT, RPT)])

    @pl.when(s == NS - 1)
    def _():
        pltpu.sync_copy(acc.at[pl.ds((NS - 1) * RPT, LAST_VALID)],
                        out.at[pl.ds(lo + (NS - 1) * RPT, LAST_VALID)])


@functools.lru_cache(maxsize=None)
def _spmm_kernel():
    return pl.kernel(
        _spmm_body,
        out_type=jax.ShapeDtypeStruct((NN, D), jnp.float32),
        mesh=_mesh(),
        scratch_types=[
            pltpu.VMEM((2, GEDGES), jnp.int32),
            pltpu.VMEM((2, GEDGES), jnp.int32),
            pltpu.VMEM((2, GEDGES), jnp.float32),
            pltpu.VMEM((NCPG, CH), jnp.int32),
            pltpu.VMEM((NCPG, CH, D), jnp.float32),
            pltpu.VMEM((ZR, D), jnp.float32),
            pltpu.VMEM_SHARED((ACC_ROWS, D), jnp.float32),
            pltpu.SemaphoreType.DMA,
            pltpu.SemaphoreType.DMA,
            pltpu.SemaphoreType.DMA,
            pltpu.SemaphoreType.DMA,
        ],
        compiler_params=pltpu.CompilerParams(use_tc_tiling_on_sc=False),
    )


UPT = B_USERS // NW
CI = 40
NIC = N_ITEMS // CI
IITERS = -(-NIC // NW)


def _build_body(light, asp, users, uidx, iidx, uout, iout,
                uv, uix, uacc, iix, iacc, sem):
    c = lax.axis_index("c")
    s = lax.axis_index("s")
    wid = s * NC + c

    ub = wid * UPT
    pltpu.sync_copy(users.at[pl.ds(ub, UPT)], uv)
    pltpu.sync_copy(uidx.at[pl.ds(ub, UPT)], uix)
    pltpu.async_copy(light.at[uv], uacc, sem).wait()
    pltpu.async_copy(asp.at[uix], uacc, sem, add=True).wait()
    pltpu.sync_copy(uacc, uout.at[pl.ds(ub, UPT)])

    def _step(k, carry):
        g = k * NW + wid

        @pl.when(g < NIC)
        def _():
            b = g * CI
            pltpu.sync_copy(light.at[pl.ds(N_USERS + b, CI)], iacc)
            pltpu.sync_copy(iidx.at[pl.ds(b, CI)], iix)
            pltpu.async_copy(asp.at[iix], iacc, sem, add=True).wait()
            pltpu.sync_copy(iacc, iout.at[pl.ds(b, CI)])

        return carry

    lax.fori_loop(0, IITERS, _step, 0)


@functools.lru_cache(maxsize=None)
def _build_kernel():
    return pl.kernel(
        _build_body,
        out_type=(jax.ShapeDtypeStruct((B_USERS, D), jnp.float32),
                  jax.ShapeDtypeStruct((N_ITEMS, D), jnp.float32)),
        mesh=_mesh(),
        scratch_types=[
            pltpu.VMEM((UPT,), jnp.int32),
            pltpu.VMEM((UPT,), jnp.int32),
            pltpu.VMEM((UPT, D), jnp.float32),
            pltpu.VMEM((CI,), jnp.int32),
            pltpu.VMEM((CI, D), jnp.float32),
            pltpu.SemaphoreType.DMA,
        ],
        compiler_params=pltpu.CompilerParams(use_tc_tiling_on_sc=False),
    )


MROWS = NN * D // 128
MB = 1000


def _mean_mlp_body(e0, e1, e2, e3, aspw, wm, bm, light, asp64):
    light[...] = (e0[...] + e1[...] + e2[...] + e3[...]) * 0.25

    @pl.when(pl.program_id(0) == 0)
    def _():
        asp64[...] = jnp.dot(aspw[...], wm[...],
                             preferred_element_type=jnp.float32) + bm[...]


def _mean_mlp(e0, e1, e2, e3, aspw, wm, bm):
    v = lambda x: x.reshape(MROWS, 128)
    light, asp64 = pl.pallas_call(
        _mean_mlp_body,
        grid=(MROWS // MB,),
        in_specs=[
            pl.BlockSpec((MB, 128), lambda i: (i, 0)),
            pl.BlockSpec((MB, 128), lambda i: (i, 0)),
            pl.BlockSpec((MB, 128), lambda i: (i, 0)),
            pl.BlockSpec((MB, 128), lambda i: (i, 0)),
            pl.BlockSpec((NQ, 384), lambda i: (0, 0)),
            pl.BlockSpec((384, D), lambda i: (0, 0)),
            pl.BlockSpec((1, D), lambda i: (0, 0)),
        ],
        out_specs=[
            pl.BlockSpec((MB, 128), lambda i: (i, 0)),
            pl.BlockSpec((NQ, D), lambda i: (0, 0)),
        ],
        out_shape=[
            jax.ShapeDtypeStruct((MROWS, 128), jnp.float32),
            jax.ShapeDtypeStruct((NQ, D), jnp.float32),
        ],
    )(v(e0), v(e1), v(e2), v(e3), aspw, wm, bm.reshape(1, D))
    return light.reshape(NN, D), asp64


IB = 512


def _rating_body(u, it, out):
    acc = lax.dot_general(u[...], it[...], (((1,), (1,)), ((), ())),
                          preferred_element_type=jnp.float32)
    out[...] = jax.nn.sigmoid(acc)


def _rating(users_emb, items_emb):
    return pl.pallas_call(
        _rating_body,
        grid=(-(-N_ITEMS // IB),),
        in_specs=[
            pl.BlockSpec((B_USERS, D), lambda i: (0, 0)),
            pl.BlockSpec((IB, D), lambda i: (i, 0)),
        ],
        out_specs=pl.BlockSpec((B_USERS, IB), lambda i: (0, i)),
        out_shape=jax.ShapeDtypeStruct((B_USERS, N_ITEMS), jnp.float32),
    )(users_emb, items_emb)


def kernel(users, edge_index, edge_weight, user_table, item_table,
           aspect_emb, W_mlp, b_mlp, user_aspect_idx, item_aspect_idx):
    users = users.astype(jnp.int32)
    src = edge_index[0]
    dst = edge_index[1]
    e0 = jnp.concatenate([user_table, item_table], axis=0)
    spmm = _spmm_kernel()
    e1 = spmm(e0, src, dst, edge_weight)
    e2 = spmm(e1, src, dst, edge_weight)
    e3 = spmm(e2, src, dst, edge_weight)
    light, asp64 = _mean_mlp(e0, e1, e2, e3, aspect_emb, W_mlp, b_mlp)
    uidx = jnp.take(user_aspect_idx, users)
    users_emb, items_emb = _build_kernel()(
        light, asp64, users, uidx, item_aspect_idx)
    return _rating(users_emb, items_emb)


# speedup vs baseline: 6.2431x; 1.1040x over previous
import functools

import jax
import jax.numpy as jnp
from jax import lax
from jax.experimental import pallas as pl
from jax.experimental.pallas import tpu as pltpu
import jax.experimental.pallas.tpu_sc as plsc

N_USERS = 25000
N_ITEMS = 25000
NN = N_USERS + N_ITEMS
D = 64
E = 800000
NQ = 512
B_USERS = 1024

NC = 2
NS = 16
NW = NC * NS
L = 16

CH = 80
EPT = E // NS
NCPG = 5
GEDGES = NCPG * CH
NG = EPT // GEDGES

HALF = NN // NC
ACC_ROWS = 25088
RPT = ACC_ROWS // NS
LAST_VALID = HALF - (NS - 1) * RPT
ZR = 28

@functools.lru_cache(maxsize=None)
def _mesh():
    return plsc.VectorSubcoreMesh(
        core_axis_name="c", subcore_axis_name="s",
        num_cores=NC, num_subcores=NS)


def _spmm_body(emb, src, dst, w, out, bsrc, bdst, bw, dstl, rows, zbuf, acc,
               esem, gsem, ssem, zsem):
    c = lax.axis_index("c")
    s = lax.axis_index("s")
    lo = c * HALF

    zro = jnp.zeros((L,), jnp.float32)

    def _zb(i, carry):
        for q in range(D // L):
            zbuf[i, pl.ds(q * L, L)] = zro
        return carry

    lax.fori_loop(0, ZR, _zb, 0)

    def _zfire(t, carry):
        pltpu.async_copy(zbuf, acc.at[pl.ds(s * RPT + t * ZR, ZR)], zsem)
        return carry

    def _zdrain(t, carry):
        pltpu.make_async_copy(
            zbuf, acc.at[pl.ds(s * RPT + t * ZR, ZR)], zsem).wait()
        return carry

    lax.fori_loop(0, RPT // ZR, _zfire, 0)
    lax.fori_loop(0, RPT // ZR, _zdrain, 0)
    plsc.subcore_barrier()

    def _eload(g, slot, fire):
        hb = s * EPT + g * GEDGES
        for buf, tab in ((bsrc, src), (bdst, dst), (bw, w)):
            if fire:
                pltpu.async_copy(tab.at[pl.ds(hb, GEDGES)], buf.at[slot],
                                 esem)
            else:
                pltpu.make_async_copy(tab.at[pl.ds(hb, GEDGES)], buf.at[slot],
                                      esem).wait()

    _eload(0, 0, True)

    def _group(g, carry):
        slot = lax.rem(g, 2)
        _eload(g, slot, False)

        @pl.when(g + 1 < NG)
        def _():
            _eload(g + 1, 1 - slot, True)

        descs = []
        for i in range(NCPG):
            @pl.when(g > 0)
            def _(i=i):
                pltpu.make_async_copy(
                    rows.at[i], acc.at[dstl.at[i]], ssem).wait()

            descs.append(
                pltpu.async_copy(emb.at[bsrc.at[slot, pl.ds(i * CH, CH)]],
                                 rows.at[i], gsem))
        for i in range(NCPG):
            descs[i].wait()
            for j in range(CH // L):
                o = i * CH + j * L
                d16 = bdst[slot, pl.ds(o, L)]
                inb = (d16 >= lo) & (d16 < lo + HALF)
                dstl[i, pl.ds(j * L, L)] = jnp.where(
                    inb, d16 - lo, HALF + (d16 & 63))
                w16 = bw[slot, pl.ds(o, L)]
                for e in range(L):
                    r = j * L + e
                    wb = w16.at[jnp.full((L,), e, jnp.int32)].get(
                        mode="promise_in_bounds")
                    for q in range(D // L):
                        rows[i, r, pl.ds(q * L, L)] = (
                            rows[i, r, pl.ds(q * L, L)] * wb)
            pltpu.async_copy(rows.at[i], acc.at[dstl.at[i]], ssem, add=True)
        return carry

    lax.fori_loop(0, NG, _group, 0)
    for i in range(NCPG):
        pltpu.make_async_copy(rows.at[i], acc.at[dstl.at[i]], ssem).wait()
    plsc.subcore_barrier()

    @pl.when(s < NS - 1)
    def _():
        pltpu.sync_copy(acc.at[pl.ds(s * RPT, RPT)],
                        out.at[pl.ds(lo + s * RPT, RPT)])

    @pl.when(s == NS - 1)
    def _():
        pltpu.sync_copy(acc.at[pl.ds((NS - 1) * RPT, LAST_VALID)],
                        out.at[pl.ds(lo + (NS - 1) * RPT, LAST_VALID)])


@functools.lru_cache(maxsize=None)
def _spmm_kernel():
    return pl.kernel(
        _spmm_body,
        out_type=jax.ShapeDtypeStruct((NN, D), jnp.float32),
        mesh=_mesh(),
        scratch_types=[
            pltpu.VMEM((2, GEDGES), jnp.int32),
            pltpu.VMEM((2, GEDGES), jnp.int32),
            pltpu.VMEM((2, GEDGES), jnp.float32),
            pltpu.VMEM((NCPG, CH), jnp.int32),
            pltpu.VMEM((NCPG, CH, D), jnp.float32),
            pltpu.VMEM((ZR, D), jnp.float32),
            pltpu.VMEM_SHARED((ACC_ROWS, D), jnp.float32),
            pltpu.SemaphoreType.DMA,
            pltpu.SemaphoreType.DMA,
            pltpu.SemaphoreType.DMA,
            pltpu.SemaphoreType.DMA,
        ],
        compiler_params=pltpu.CompilerParams(use_tc_tiling_on_sc=False),
    )


UPT = B_USERS // NW
CI = 40
NIC = N_ITEMS // CI
IITERS = -(-NIC // NW)


def _build_body(light, asp, users, uidx, iidx, uout, iout,
                uv, uix, uacc, iix, iacc, sem):
    c = lax.axis_index("c")
    s = lax.axis_index("s")
    wid = s * NC + c

    ub = wid * UPT
    pltpu.sync_copy(users.at[pl.ds(ub, UPT)], uv)
    pltpu.sync_copy(uidx.at[pl.ds(ub, UPT)], uix)
    pltpu.async_copy(light.at[uv], uacc, sem).wait()
    pltpu.async_copy(asp.at[uix], uacc, sem, add=True).wait()
    pltpu.sync_copy(uacc, uout.at[pl.ds(ub, UPT)])

    def _step(k, carry):
        g = k * NW + wid

        @pl.when(g < NIC)
        def _():
            b = g * CI
            pltpu.sync_copy(light.at[pl.ds(N_USERS + b, CI)], iacc)
            pltpu.sync_copy(iidx.at[pl.ds(b, CI)], iix)
            pltpu.async_copy(asp.at[iix], iacc, sem, add=True).wait()
            pltpu.sync_copy(iacc, iout.at[pl.ds(b, CI)])

        return carry

    lax.fori_loop(0, IITERS, _step, 0)


@functools.lru_cache(maxsize=None)
def _build_kernel():
    return pl.kernel(
        _build_body,
        out_type=(jax.ShapeDtypeStruct((B_USERS, D), jnp.float32),
                  jax.ShapeDtypeStruct((N_ITEMS, D), jnp.float32)),
        mesh=_mesh(),
        scratch_types=[
            pltpu.VMEM((UPT,), jnp.int32),
            pltpu.VMEM((UPT,), jnp.int32),
            pltpu.VMEM((UPT, D), jnp.float32),
            pltpu.VMEM((CI,), jnp.int32),
            pltpu.VMEM((CI, D), jnp.float32),
            pltpu.SemaphoreType.DMA,
        ],
        compiler_params=pltpu.CompilerParams(use_tc_tiling_on_sc=False),
    )


MROWS = NN * D // 128
MB = 1000


def _mean_mlp_body(e0, e1, e2, e3, aspw, wm, bm, light, asp64):
    light[...] = (e0[...] + e1[...] + e2[...] + e3[...]) * 0.25

    @pl.when(pl.program_id(0) == 0)
    def _():
        asp64[...] = jnp.dot(aspw[...], wm[...],
                             preferred_element_type=jnp.float32) + bm[...]


def _mean_mlp(e0, e1, e2, e3, aspw, wm, bm):
    v = lambda x: x.reshape(MROWS, 128)
    light, asp64 = pl.pallas_call(
        _mean_mlp_body,
        grid=(MROWS // MB,),
        in_specs=[
            pl.BlockSpec((MB, 128), lambda i: (i, 0)),
            pl.BlockSpec((MB, 128), lambda i: (i, 0)),
            pl.BlockSpec((MB, 128), lambda i: (i, 0)),
            pl.BlockSpec((MB, 128), lambda i: (i, 0)),
            pl.BlockSpec((NQ, 384), lambda i: (0, 0)),
            pl.BlockSpec((384, D), lambda i: (0, 0)),
            pl.BlockSpec((1, D), lambda i: (0, 0)),
        ],
        out_specs=[
            pl.BlockSpec((MB, 128), lambda i: (i, 0)),
            pl.BlockSpec((NQ, D), lambda i: (0, 0)),
        ],
        out_shape=[
            jax.ShapeDtypeStruct((MROWS, 128), jnp.float32),
            jax.ShapeDtypeStruct((NQ, D), jnp.float32),
        ],
    )(v(e0), v(e1), v(e2), v(e3), aspw, wm, bm.reshape(1, D))
    return light.reshape(NN, D), asp64


IB = 512


def _rating_body(u, it, out):
    acc = lax.dot_general(u[...], it[...], (((1,), (1,)), ((), ())),
                          preferred_element_type=jnp.float32)
    out[...] = jax.nn.sigmoid(acc)


def _rating(users_emb, items_emb):
    return pl.pallas_call(
        _rating_body,
        grid=(-(-N_ITEMS // IB),),
        in_specs=[
            pl.BlockSpec((B_USERS, D), lambda i: (0, 0)),
            pl.BlockSpec((IB, D), lambda i: (i, 0)),
        ],
        out_specs=pl.BlockSpec((B_USERS, IB), lambda i: (0, i)),
        out_shape=jax.ShapeDtypeStruct((B_USERS, N_ITEMS), jnp.float32),
    )(users_emb, items_emb)


def kernel(users, edge_index, edge_weight, user_table, item_table,
           aspect_emb, W_mlp, b_mlp, user_aspect_idx, item_aspect_idx):
    users = users.astype(jnp.int32)
    src = edge_index[0]
    dst = edge_index[1]
    e0 = jnp.concatenate([user_table, item_table], axis=0)
    spmm = _spmm_kernel()
    e1 = spmm(e0, src, dst, edge_weight)
    e2 = spmm(e1, src, dst, edge_weight)
    e3 = spmm(e2, src, dst, edge_weight)
    light, asp64 = _mean_mlp(e0, e1, e2, e3, aspect_emb, W_mlp, b_mlp)
    uidx = jnp.take(user_aspect_idx, users)
    users_emb, items_emb = _build_kernel()(
        light, asp64, users, uidx, item_aspect_idx)
    return _rating(users_emb, items_emb)
